# baseline (speedup 1.0000x reference)
import functools

import jax
import jax.numpy as jnp
from jax import lax
from jax.experimental import pallas as pl
from jax.experimental.pallas import tpu as pltpu
from jax.experimental.pallas import tpu_sc as plsc

D = 1024
H = 4096
E = 8
T = 8192
CAP = T // E
NSLOT = E * CAP
DUMP = NSLOT

BT = 1024
HB = 2048
NH = H // HB

NC = 2
NS = 16
NW = NC * NS
LANES = 16

ROWS_PER_W = T // NW
CHUNK = 32


def _route_body(x_ref, wg_ref, bg_ref, slot_ref, scale_ref, cnt_ref):
    pi = pl.program_id(0)

    @pl.when(pi == 0)
    def _():
        cnt_ref[...] = jnp.zeros((1, E), jnp.int32)

    x = x_ref[...]
    logits = lax.dot_general(
        x, wg_ref[...], (((1,), (0,)), ((), ())),
        preferred_element_type=jnp.float32,
    ) + bg_ref[...]

    m = jnp.max(logits, axis=1, keepdims=True)
    p = jnp.exp(logits - m)
    denom = jnp.sum(p, axis=1, keepdims=True)
    gate = 1.0 / denom

    idx = jnp.argmax(logits, axis=1)[:, None].astype(jnp.int32)
    lane = lax.broadcasted_iota(jnp.int32, (BT, E), 1)
    oh = (lane == idx).astype(jnp.float32)
    SB = 256
    row = lax.broadcasted_iota(jnp.int32, (SB, SB), 0)
    col = lax.broadcasted_iota(jnp.int32, (SB, SB), 1)
    ltri = (row > col).astype(jnp.bfloat16)
    offs = cnt_ref[...].astype(jnp.float32)
    ranks = []
    for k in range(BT // SB):
        ohk = oh[k * SB:(k + 1) * SB]
        csub = lax.dot_general(ltri, ohk.astype(jnp.bfloat16),
                               (((1,), (0,)), ((), ())),
                               preferred_element_type=jnp.float32)
        rk = (jnp.sum(csub * ohk, axis=1, keepdims=True)
              + jnp.sum(ohk * offs, axis=1, keepdims=True))
        ranks.append(rk)
        offs = offs + jnp.sum(ohk, axis=0, keepdims=True)
    rank = jnp.concatenate(ranks, axis=0).astype(jnp.int32)
    cnt_ref[...] = offs.astype(jnp.int32)

    keep = rank < CAP
    slot_ref[...] = jnp.where(keep, idx * CAP + rank, DUMP)
    scale_ref[...] = jnp.where(keep, gate, 0.0)


def _route(xf, Wg, bg):
    return pl.pallas_call(
        _route_body,
        grid=(T // BT,),
        in_specs=[
            pl.BlockSpec((BT, D), lambda i: (i, 0)),
            pl.BlockSpec((D, E), lambda i: (0, 0)),
            pl.BlockSpec((1, E), lambda i: (0, 0)),
        ],
        out_specs=[
            pl.BlockSpec((BT, 1), lambda i: (i, 0)),
            pl.BlockSpec((BT, 1), lambda i: (i, 0)),
        ],
        out_shape=[
            jax.ShapeDtypeStruct((T, 1), jnp.int32),
            jax.ShapeDtypeStruct((T, 1), jnp.float32),
        ],
        scratch_shapes=[pltpu.VMEM((1, E), jnp.int32)],
        compiler_params=pltpu.CompilerParams(
            dimension_semantics=("arbitrary",),
        ),
    )(xf, Wg, bg.reshape(1, E))


def _invert(slot, scale):
    mesh = plsc.VectorSubcoreMesh(core_axis_name="c", subcore_axis_name="s")

    @functools.partial(
        pl.kernel,
        mesh=mesh,
        out_type=[
            jax.ShapeDtypeStruct((NSLOT,), jnp.int32),
            jax.ShapeDtypeStruct((NSLOT,), jnp.float32),
        ],
        scratch_types=[
            pltpu.VMEM((T,), jnp.int32),
            pltpu.VMEM((T,), jnp.float32),
            pltpu.VMEM((NSLOT,), jnp.int32),
            pltpu.VMEM((NSLOT,), jnp.float32),
        ],
        compiler_params=pltpu.CompilerParams(needs_layout_passes=False),
    )
    def k(slot_hbm, scale_hbm, src_hbm, sscale_hbm, slot_v, scale_v,
          src_v, sscale_v):
        wid = lax.axis_index("c") * NS + lax.axis_index("s")

        @pl.when(wid == 0)
        def _():
            pltpu.sync_copy(slot_hbm, slot_v)
            pltpu.sync_copy(scale_hbm, scale_v)
            zero_i = jnp.zeros((LANES,), jnp.int32)
            zero_f = jnp.zeros((LANES,), jnp.float32)

            def init(i, _):
                src_v[pl.ds(i * LANES, LANES)] = zero_i
                sscale_v[pl.ds(i * LANES, LANES)] = zero_f
                return 0

            lax.fori_loop(0, NSLOT // LANES, init, 0)

            tbase = lax.iota(jnp.int32, LANES)

            def scat(i, _):
                s = slot_v[pl.ds(i * LANES, LANES)]
                tok = tbase + i * LANES
                plsc.store_scatter(src_v, [s], tok, mask=s < NSLOT)
                return 0

            lax.fori_loop(0, T // LANES, scat, 0)

            def gath(i, _):
                sv = plsc.load_gather(scale_v,
                                      [src_v[pl.ds(i * LANES, LANES)]])
                sscale_v[pl.ds(i * LANES, LANES)] = sv
                return 0

            lax.fori_loop(0, NSLOT // LANES, gath, 0)

            pltpu.sync_copy(src_v, src_hbm)
            pltpu.sync_copy(sscale_v, sscale_hbm)

    return k(slot, scale)


def _gather_rows(table, idx, base=0, nrows=None):
    n = nrows if nrows is not None else idx.shape[0]
    mesh = plsc.VectorSubcoreMesh(core_axis_name="c", subcore_axis_name="s")

    rpw = n // NW
    nchunk = rpw // CHUNK

    @functools.partial(
        pl.kernel,
        mesh=mesh,
        out_type=jax.ShapeDtypeStruct((n, D), jnp.float32),
        scratch_types=[
            pltpu.VMEM((2, CHUNK), jnp.int32),
            pltpu.VMEM((2, CHUNK, D), jnp.float32),
            [pltpu.SemaphoreType.DMA] * 2,
            [pltpu.SemaphoreType.DMA] * 2,
        ],
        compiler_params=pltpu.CompilerParams(needs_layout_passes=False),
    )
    def k(table_hbm, idx_hbm, out_hbm, idx_v, rows_v, gsem, wsem):
        wid = lax.axis_index("c") * NS + lax.axis_index("s")

        def start_gather(c):
            b = c % 2
            lb = wid * rpw + c * CHUNK
            pltpu.sync_copy(idx_hbm.at[pl.ds(base + lb, CHUNK)], idx_v.at[b])
            return pltpu.async_copy(table_hbm.at[idx_v.at[b]], rows_v.at[b],
                                    gsem[b])

        gh = [None, None]
        wh = [None, None]
        gh[0] = start_gather(0)
        for c in range(nchunk):
            b = c % 2
            gh[b].wait()
            if c + 1 < nchunk:
                if wh[(c + 1) % 2] is not None:
                    wh[(c + 1) % 2].wait()
                gh[(c + 1) % 2] = start_gather(c + 1)
            lb = wid * rpw + c * CHUNK
            wh[b] = pltpu.async_copy(rows_v.at[b],
                                     out_hbm.at[pl.ds(lb, CHUNK)], wsem[b])
        wh[(nchunk - 1) % 2].wait()
        if nchunk >= 2:
            wh[(nchunk - 2) % 2].wait()

    return k(table, idx)


def _make_mlp_body(ne, zero_block, aliased):
    def body(xs_ref, w1_ref, b1_ref, w2_ref, b2_ref, ss_ref, *rest):
        if aliased:
            _, out_ref, acc_ref = rest
        else:
            out_ref, acc_ref = rest
        e = pl.program_id(0)
        h = pl.program_id(1)

        @pl.when(jnp.logical_and(e < ne, h == 0))
        def _():
            acc_ref[...] = jnp.zeros_like(acc_ref)

        @pl.when(e < ne)
        def _():
            xb = xs_ref[0].astype(jnp.bfloat16)
            hpre = lax.dot_general(
                xb, w1_ref[0].astype(jnp.bfloat16), (((1,), (0,)), ((), ())),
                preferred_element_type=jnp.float32) + b1_ref[0]
            hrelu = jnp.maximum(hpre, 0.0).astype(jnp.bfloat16)
            acc_ref[...] += lax.dot_general(
                hrelu, w2_ref[0].astype(jnp.bfloat16), (((1,), (0,)), ((), ())),
                preferred_element_type=jnp.float32)

        @pl.when(h == NH - 1)
        def _():
            @pl.when(e < ne)
            def _():
                out_ref[0] = (acc_ref[...] + b2_ref[0]) * ss_ref[0]

            if zero_block:
                @pl.when(e == ne)
                def _():
                    out_ref[0] = jnp.zeros_like(out_ref[0])

    return body


def _mlp_part(xs_half, W1, b1, W2, b2, sscale, e0, ne, zero_block, init):
    ng = ne + (1 if zero_block else 0)
    in_specs = [
        pl.BlockSpec((1, CAP, D), lambda e, h: (jnp.minimum(e, ne - 1), 0, 0)),
        pl.BlockSpec((1, D, HB),
                     lambda e, h: (e0 + jnp.minimum(e, ne - 1), 0, h)),
        pl.BlockSpec((1, 1, HB),
                     lambda e, h: (e0 + jnp.minimum(e, ne - 1), 0, h)),
        pl.BlockSpec((1, HB, D),
                     lambda e, h: (e0 + jnp.minimum(e, ne - 1), h, 0)),
        pl.BlockSpec((1, 1, D),
                     lambda e, h: (e0 + jnp.minimum(e, ne - 1), 0, 0)),
        pl.BlockSpec((1, CAP, 1),
                     lambda e, h: (e0 + jnp.minimum(e, ne - 1), 0, 0)),
    ]
    args = [xs_half.reshape(ne, CAP, D), W1, b1.reshape(E, 1, H), W2,
            b2.reshape(E, 1, D), sscale.reshape(E, CAP, 1)]
    io_aliases = {}
    if init is not None:
        in_specs.append(pl.BlockSpec(memory_space=pl.ANY))
        args.append(init)
        io_aliases = {6: 0}
    if zero_block:
        out_map = lambda e, h: (jnp.where(e < ne, e0 + e, E), 0, 0)
    else:
        out_map = lambda e, h: (e0 + e, 0, 0)
    return pl.pallas_call(
        _make_mlp_body(ne, zero_block, init is not None),
        grid=(ng, NH),
        in_specs=in_specs,
        out_specs=pl.BlockSpec((1, CAP, D), out_map),
        out_shape=jax.ShapeDtypeStruct((E + 1, CAP, D), jnp.float32),
        scratch_shapes=[pltpu.VMEM((CAP, D), jnp.float32)],
        input_output_aliases=io_aliases,
        compiler_params=pltpu.CompilerParams(
            dimension_semantics=("arbitrary", "arbitrary"),
            vmem_limit_bytes=100 * 1024 * 1024,
        ),
    )(*args)


def kernel(x, Wg, bg, W1, b1, W2, b2):
    orig_shape = x.shape
    xf = x.reshape(T, D)

    slot, scale = _route(xf, Wg, bg)
    slot = slot.reshape(T)
    scale = scale.reshape(T)

    src, sscale = _invert(slot, scale)
    half = E // 2
    xs_a = _gather_rows(xf, src, 0, T // 2)
    xs_b = _gather_rows(xf, src, T // 2, T // 2)
    yb_a = _mlp_part(xs_a, W1, b1, W2, b2, sscale, 0, half, True, None)
    yb = _mlp_part(xs_b, W1, b1, W2, b2, sscale, half, half, False, yb_a)
    y = _gather_rows(yb.reshape((E + 1) * CAP, D), slot)
    return y.reshape(orig_shape)


# speedup vs baseline: 1.0068x; 1.0068x over previous
import functools

import jax
import jax.numpy as jnp
from jax import lax
from jax.experimental import pallas as pl
from jax.experimental.pallas import tpu as pltpu
from jax.experimental.pallas import tpu_sc as plsc

D = 1024
H = 4096
E = 8
T = 8192
CAP = T // E
NSLOT = E * CAP
DUMP = NSLOT

BT = 2048
HB = 2048
NH = H // HB

NC = 2
NS = 16
NW = NC * NS
LANES = 16

ROWS_PER_W = T // NW
CHUNK = 32


def _route_body(x_ref, wg_ref, bg_ref, slot_ref, scale_ref, cnt_ref):
    pi = pl.program_id(0)

    @pl.when(pi == 0)
    def _():
        cnt_ref[...] = jnp.zeros((1, E), jnp.int32)

    x = x_ref[...]
    logits = lax.dot_general(
        x, wg_ref[...], (((1,), (0,)), ((), ())),
        preferred_element_type=jnp.float32,
    ) + bg_ref[...]

    m = jnp.max(logits, axis=1, keepdims=True)
    p = jnp.exp(logits - m)
    denom = jnp.sum(p, axis=1, keepdims=True)
    gate = 1.0 / denom

    idx = jnp.argmax(logits, axis=1)[:, None].astype(jnp.int32)
    lane = lax.broadcasted_iota(jnp.int32, (BT, E), 1)
    oh = (lane == idx).astype(jnp.float32)
    SB = 256
    row = lax.broadcasted_iota(jnp.int32, (SB, SB), 0)
    col = lax.broadcasted_iota(jnp.int32, (SB, SB), 1)
    ltri = (row > col).astype(jnp.bfloat16)
    offs = cnt_ref[...].astype(jnp.float32)
    ranks = []
    for k in range(BT // SB):
        ohk = oh[k * SB:(k + 1) * SB]
        csub = lax.dot_general(ltri, ohk.astype(jnp.bfloat16),
                               (((1,), (0,)), ((), ())),
                               preferred_element_type=jnp.float32)
        rk = (jnp.sum(csub * ohk, axis=1, keepdims=True)
              + jnp.sum(ohk * offs, axis=1, keepdims=True))
        ranks.append(rk)
        offs = offs + jnp.sum(ohk, axis=0, keepdims=True)
    rank = jnp.concatenate(ranks, axis=0).astype(jnp.int32)
    cnt_ref[...] = offs.astype(jnp.int32)

    keep = rank < CAP
    slot_ref[...] = jnp.where(keep, idx * CAP + rank, DUMP)
    scale_ref[...] = jnp.where(keep, gate, 0.0)


def _route(xf, Wg, bg):
    return pl.pallas_call(
        _route_body,
        grid=(T // BT,),
        in_specs=[
            pl.BlockSpec((BT, D), lambda i: (i, 0)),
            pl.BlockSpec((D, E), lambda i: (0, 0)),
            pl.BlockSpec((1, E), lambda i: (0, 0)),
        ],
        out_specs=[
            pl.BlockSpec((BT, 1), lambda i: (i, 0)),
            pl.BlockSpec((BT, 1), lambda i: (i, 0)),
        ],
        out_shape=[
            jax.ShapeDtypeStruct((T, 1), jnp.int32),
            jax.ShapeDtypeStruct((T, 1), jnp.float32),
        ],
        scratch_shapes=[pltpu.VMEM((1, E), jnp.int32)],
        compiler_params=pltpu.CompilerParams(
            dimension_semantics=("arbitrary",),
        ),
    )(xf, Wg, bg.reshape(1, E))


def _invert(slot, scale):
    mesh = plsc.VectorSubcoreMesh(core_axis_name="c", subcore_axis_name="s")

    @functools.partial(
        pl.kernel,
        mesh=mesh,
        out_type=[
            jax.ShapeDtypeStruct((NSLOT,), jnp.int32),
            jax.ShapeDtypeStruct((NSLOT,), jnp.float32),
        ],
        scratch_types=[
            pltpu.VMEM((T,), jnp.int32),
            pltpu.VMEM((T,), jnp.float32),
            pltpu.VMEM((NSLOT,), jnp.int32),
            pltpu.VMEM((NSLOT,), jnp.float32),
        ],
        compiler_params=pltpu.CompilerParams(needs_layout_passes=False),
    )
    def k(slot_hbm, scale_hbm, src_hbm, sscale_hbm, slot_v, scale_v,
          src_v, sscale_v):
        wid = lax.axis_index("c") * NS + lax.axis_index("s")

        @pl.when(wid == 0)
        def _():
            pltpu.sync_copy(slot_hbm, slot_v)
            pltpu.sync_copy(scale_hbm, scale_v)
            zero_i = jnp.zeros((LANES,), jnp.int32)
            zero_f = jnp.zeros((LANES,), jnp.float32)

            def init(i, _):
                src_v[pl.ds(i * LANES, LANES)] = zero_i
                sscale_v[pl.ds(i * LANES, LANES)] = zero_f
                return 0

            lax.fori_loop(0, NSLOT // LANES, init, 0)

            tbase = lax.iota(jnp.int32, LANES)

            def scat(i, _):
                s = slot_v[pl.ds(i * LANES, LANES)]
                tok = tbase + i * LANES
                plsc.store_scatter(src_v, [s], tok, mask=s < NSLOT)
                return 0

            lax.fori_loop(0, T // LANES, scat, 0)

            def gath(i, _):
                sv = plsc.load_gather(scale_v,
                                      [src_v[pl.ds(i * LANES, LANES)]])
                sscale_v[pl.ds(i * LANES, LANES)] = sv
                return 0

            lax.fori_loop(0, NSLOT // LANES, gath, 0)

            pltpu.sync_copy(src_v, src_hbm)
            pltpu.sync_copy(sscale_v, sscale_hbm)

    return k(slot, scale)


def _gather_rows(table, idx, base=0, nrows=None):
    n = nrows if nrows is not None else idx.shape[0]
    mesh = plsc.VectorSubcoreMesh(core_axis_name="c", subcore_axis_name="s")

    rpw = n // NW
    nchunk = rpw // CHUNK

    @functools.partial(
        pl.kernel,
        mesh=mesh,
        out_type=jax.ShapeDtypeStruct((n, D), jnp.float32),
        scratch_types=[
            pltpu.VMEM((2, CHUNK), jnp.int32),
            pltpu.VMEM((2, CHUNK, D), jnp.float32),
            [pltpu.SemaphoreType.DMA] * 2,
            [pltpu.SemaphoreType.DMA] * 2,
        ],
        compiler_params=pltpu.CompilerParams(needs_layout_passes=False),
    )
    def k(table_hbm, idx_hbm, out_hbm, idx_v, rows_v, gsem, wsem):
        wid = lax.axis_index("c") * NS + lax.axis_index("s")

        def start_gather(c):
            b = c % 2
            lb = wid * rpw + c * CHUNK
            pltpu.sync_copy(idx_hbm.at[pl.ds(base + lb, CHUNK)], idx_v.at[b])
            return pltpu.async_copy(table_hbm.at[idx_v.at[b]], rows_v.at[b],
                                    gsem[b])

        gh = [None, None]
        wh = [None, None]
        gh[0] = start_gather(0)
        for c in range(nchunk):
            b = c % 2
            gh[b].wait()
            if c + 1 < nchunk:
                if wh[(c + 1) % 2] is not None:
                    wh[(c + 1) % 2].wait()
                gh[(c + 1) % 2] = start_gather(c + 1)
            lb = wid * rpw + c * CHUNK
            wh[b] = pltpu.async_copy(rows_v.at[b],
                                     out_hbm.at[pl.ds(lb, CHUNK)], wsem[b])
        wh[(nchunk - 1) % 2].wait()
        if nchunk >= 2:
            wh[(nchunk - 2) % 2].wait()

    return k(table, idx)


def _make_mlp_body(ne, zero_block, aliased):
    def body(xs_ref, w1_ref, b1_ref, w2_ref, b2_ref, ss_ref, *rest):
        if aliased:
            _, out_ref, acc_ref = rest
        else:
            out_ref, acc_ref = rest
        e = pl.program_id(0)
        h = pl.program_id(1)

        @pl.when(jnp.logical_and(e < ne, h == 0))
        def _():
            acc_ref[...] = jnp.zeros_like(acc_ref)

        @pl.when(e < ne)
        def _():
            xb = xs_ref[0].astype(jnp.bfloat16)
            hpre = lax.dot_general(
                xb, w1_ref[0].astype(jnp.bfloat16), (((1,), (0,)), ((), ())),
                preferred_element_type=jnp.float32) + b1_ref[0]
            hrelu = jnp.maximum(hpre, 0.0).astype(jnp.bfloat16)
            acc_ref[...] += lax.dot_general(
                hrelu, w2_ref[0].astype(jnp.bfloat16), (((1,), (0,)), ((), ())),
                preferred_element_type=jnp.float32)

        @pl.when(h == NH - 1)
        def _():
            @pl.when(e < ne)
            def _():
                out_ref[0] = (acc_ref[...] + b2_ref[0]) * ss_ref[0]

            if zero_block:
                @pl.when(e == ne)
                def _():
                    out_ref[0] = jnp.zeros_like(out_ref[0])

    return body


def _mlp_part(xs_half, W1, b1, W2, b2, sscale, e0, ne, zero_block, init):
    ng = ne + (1 if zero_block else 0)
    in_specs = [
        pl.BlockSpec((1, CAP, D), lambda e, h: (jnp.minimum(e, ne - 1), 0, 0)),
        pl.BlockSpec((1, D, HB),
                     lambda e, h: (e0 + jnp.minimum(e, ne - 1), 0, h)),
        pl.BlockSpec((1, 1, HB),
                     lambda e, h: (e0 + jnp.minimum(e, ne - 1), 0, h)),
        pl.BlockSpec((1, HB, D),
                     lambda e, h: (e0 + jnp.minimum(e, ne - 1), h, 0)),
        pl.BlockSpec((1, 1, D),
                     lambda e, h: (e0 + jnp.minimum(e, ne - 1), 0, 0)),
        pl.BlockSpec((1, CAP, 1),
                     lambda e, h: (e0 + jnp.minimum(e, ne - 1), 0, 0)),
    ]
    args = [xs_half.reshape(ne, CAP, D), W1, b1.reshape(E, 1, H), W2,
            b2.reshape(E, 1, D), sscale.reshape(E, CAP, 1)]
    io_aliases = {}
    if init is not None:
        in_specs.append(pl.BlockSpec(memory_space=pl.ANY))
        args.append(init)
        io_aliases = {6: 0}
    if zero_block:
        out_map = lambda e, h: (jnp.where(e < ne, e0 + e, E), 0, 0)
    else:
        out_map = lambda e, h: (e0 + e, 0, 0)
    return pl.pallas_call(
        _make_mlp_body(ne, zero_block, init is not None),
        grid=(ng, NH),
        in_specs=in_specs,
        out_specs=pl.BlockSpec((1, CAP, D), out_map),
        out_shape=jax.ShapeDtypeStruct((E + 1, CAP, D), jnp.float32),
        scratch_shapes=[pltpu.VMEM((CAP, D), jnp.float32)],
        input_output_aliases=io_aliases,
        compiler_params=pltpu.CompilerParams(
            dimension_semantics=("arbitrary", "arbitrary"),
            vmem_limit_bytes=100 * 1024 * 1024,
        ),
    )(*args)


def kernel(x, Wg, bg, W1, b1, W2, b2):
    orig_shape = x.shape
    xf = x.reshape(T, D)

    slot, scale = _route(xf, Wg, bg)
    slot = slot.reshape(T)
    scale = scale.reshape(T)

    src, sscale = _invert(slot, scale)
    xs = _gather_rows(xf, src)
    yb = _mlp_part(xs, W1, b1, W2, b2, sscale, 0, E, True, None)
    y = _gather_rows(yb.reshape((E + 1) * CAP, D), slot)
    return y.reshape(orig_shape)


# speedup vs baseline: 1.0233x; 1.0164x over previous
import functools

import jax
import jax.numpy as jnp
from jax import lax
from jax.experimental import pallas as pl
from jax.experimental.pallas import tpu as pltpu
from jax.experimental.pallas import tpu_sc as plsc

D = 1024
H = 4096
E = 8
T = 8192
CAP = T // E
NSLOT = E * CAP
DUMP = NSLOT

BT = 2048
HB = 2048
NH = H // HB

NC = 2
NS = 16
NW = NC * NS
LANES = 16

ROWS_PER_W = T // NW
CHUNK = 32


def _route_body(x_ref, wg_ref, bg_ref, slot_ref, scale_ref, cnt_ref):
    pi = pl.program_id(0)

    @pl.when(pi == 0)
    def _():
        cnt_ref[...] = jnp.zeros((1, E), jnp.int32)

    x = x_ref[...]
    logits = lax.dot_general(
        x, wg_ref[...], (((1,), (0,)), ((), ())),
        preferred_element_type=jnp.float32,
    ) + bg_ref[...]

    m = jnp.max(logits, axis=1, keepdims=True)
    p = jnp.exp(logits - m)
    denom = jnp.sum(p, axis=1, keepdims=True)
    gate = 1.0 / denom

    idx = jnp.argmax(logits, axis=1)[:, None].astype(jnp.int32)
    lane = lax.broadcasted_iota(jnp.int32, (BT, E), 1)
    oh = (lane == idx).astype(jnp.float32)
    SB = 256
    row = lax.broadcasted_iota(jnp.int32, (SB, SB), 0)
    col = lax.broadcasted_iota(jnp.int32, (SB, SB), 1)
    ltri = (row > col).astype(jnp.bfloat16)
    offs = cnt_ref[...].astype(jnp.float32)
    ranks = []
    for k in range(BT // SB):
        ohk = oh[k * SB:(k + 1) * SB]
        csub = lax.dot_general(ltri, ohk.astype(jnp.bfloat16),
                               (((1,), (0,)), ((), ())),
                               preferred_element_type=jnp.float32)
        rk = (jnp.sum(csub * ohk, axis=1, keepdims=True)
              + jnp.sum(ohk * offs, axis=1, keepdims=True))
        ranks.append(rk)
        offs = offs + jnp.sum(ohk, axis=0, keepdims=True)
    rank = jnp.concatenate(ranks, axis=0).astype(jnp.int32)
    cnt_ref[...] = offs.astype(jnp.int32)

    keep = rank < CAP
    slot_ref[...] = jnp.where(keep, idx * CAP + rank, DUMP)
    scale_ref[...] = jnp.where(keep, gate, 0.0)


def _route(xf, Wg, bg):
    return pl.pallas_call(
        _route_body,
        grid=(T // BT,),
        in_specs=[
            pl.BlockSpec((BT, D), lambda i: (i, 0)),
            pl.BlockSpec((D, E), lambda i: (0, 0)),
            pl.BlockSpec((1, E), lambda i: (0, 0)),
        ],
        out_specs=[
            pl.BlockSpec((BT, 1), lambda i: (i, 0)),
            pl.BlockSpec((BT, 1), lambda i: (i, 0)),
        ],
        out_shape=[
            jax.ShapeDtypeStruct((T, 1), jnp.int32),
            jax.ShapeDtypeStruct((T, 1), jnp.float32),
        ],
        scratch_shapes=[pltpu.VMEM((1, E), jnp.int32)],
        compiler_params=pltpu.CompilerParams(
            dimension_semantics=("arbitrary",),
        ),
    )(xf, Wg, bg.reshape(1, E))


def _invert(slot, scale):
    mesh = plsc.VectorSubcoreMesh(core_axis_name="c", subcore_axis_name="s")

    @functools.partial(
        pl.kernel,
        mesh=mesh,
        out_type=[
            jax.ShapeDtypeStruct((NSLOT,), jnp.int32),
            jax.ShapeDtypeStruct((NSLOT,), jnp.float32),
        ],
        scratch_types=[
            pltpu.VMEM((T,), jnp.int32),
            pltpu.VMEM((T,), jnp.float32),
            pltpu.VMEM((NSLOT,), jnp.int32),
            pltpu.VMEM((NSLOT,), jnp.float32),
        ],
        compiler_params=pltpu.CompilerParams(needs_layout_passes=False),
    )
    def k(slot_hbm, scale_hbm, src_hbm, sscale_hbm, slot_v, scale_v,
          src_v, sscale_v):
        wid = lax.axis_index("c") * NS + lax.axis_index("s")

        @pl.when(wid == 0)
        def _():
            pltpu.sync_copy(slot_hbm, slot_v)
            pltpu.sync_copy(scale_hbm, scale_v)
            zero_i = jnp.zeros((LANES,), jnp.int32)
            zero_f = jnp.zeros((LANES,), jnp.float32)

            def init(i, _):
                src_v[pl.ds(i * LANES, LANES)] = zero_i
                sscale_v[pl.ds(i * LANES, LANES)] = zero_f
                return 0

            lax.fori_loop(0, NSLOT // LANES, init, 0)

            tbase = lax.iota(jnp.int32, LANES)

            def scat(i, _):
                s = slot_v[pl.ds(i * LANES, LANES)]
                tok = tbase + i * LANES
                plsc.store_scatter(src_v, [s], tok, mask=s < NSLOT)
                return 0

            lax.fori_loop(0, T // LANES, scat, 0)

            def gath(i, _):
                sv = plsc.load_gather(scale_v,
                                      [src_v[pl.ds(i * LANES, LANES)]])
                sscale_v[pl.ds(i * LANES, LANES)] = sv
                return 0

            lax.fori_loop(0, NSLOT // LANES, gath, 0)

            pltpu.sync_copy(src_v, src_hbm)
            pltpu.sync_copy(sscale_v, sscale_hbm)

    return k(slot, scale)


def _gather_rows(table, idx, base=0, nrows=None):
    n = nrows if nrows is not None else idx.shape[0]
    mesh = plsc.VectorSubcoreMesh(core_axis_name="c", subcore_axis_name="s")

    rpw = n // NW
    nchunk = rpw // CHUNK

    @functools.partial(
        pl.kernel,
        mesh=mesh,
        out_type=jax.ShapeDtypeStruct((n, D), jnp.float32),
        scratch_types=[
            pltpu.VMEM((2, CHUNK), jnp.int32),
            pltpu.VMEM((2, CHUNK, D), jnp.float32),
            [pltpu.SemaphoreType.DMA] * 2,
            [pltpu.SemaphoreType.DMA] * 2,
        ],
        compiler_params=pltpu.CompilerParams(needs_layout_passes=False),
    )
    def k(table_hbm, idx_hbm, out_hbm, idx_v, rows_v, gsem, wsem):
        wid = lax.axis_index("c") * NS + lax.axis_index("s")

        def start_gather(c):
            b = c % 2
            lb = wid * rpw + c * CHUNK
            pltpu.sync_copy(idx_hbm.at[pl.ds(base + lb, CHUNK)], idx_v.at[b])
            return pltpu.async_copy(table_hbm.at[idx_v.at[b]], rows_v.at[b],
                                    gsem[b])

        gh = [None, None]
        wh = [None, None]
        gh[0] = start_gather(0)
        for c in range(nchunk):
            b = c % 2
            gh[b].wait()
            if c + 1 < nchunk:
                if wh[(c + 1) % 2] is not None:
                    wh[(c + 1) % 2].wait()
                gh[(c + 1) % 2] = start_gather(c + 1)
            lb = wid * rpw + c * CHUNK
            wh[b] = pltpu.async_copy(rows_v.at[b],
                                     out_hbm.at[pl.ds(lb, CHUNK)], wsem[b])
        wh[(nchunk - 1) % 2].wait()
        if nchunk >= 2:
            wh[(nchunk - 2) % 2].wait()

    return k(table, idx)


def _make_mlp_body(ne, zero_block, aliased):
    def body(xs_ref, w1_ref, b1_ref, w2_ref, b2_ref, ss_ref, *rest):
        if aliased:
            _, out_ref, acc_ref = rest
        else:
            out_ref, acc_ref = rest
        e = pl.program_id(0)
        h = pl.program_id(1)

        @pl.when(jnp.logical_and(e < ne, h == 0))
        def _():
            acc_ref[...] = jnp.zeros_like(acc_ref)

        @pl.when(e < ne)
        def _():
            xb = xs_ref[0].astype(jnp.bfloat16)
            hpre = lax.dot_general(
                xb, w1_ref[0].astype(jnp.bfloat16), (((1,), (0,)), ((), ())),
                preferred_element_type=jnp.float32) + b1_ref[0]
            hrelu = jnp.maximum(hpre, 0.0).astype(jnp.bfloat16)
            acc_ref[...] += lax.dot_general(
                hrelu, w2_ref[0].astype(jnp.bfloat16), (((1,), (0,)), ((), ())),
                preferred_element_type=jnp.float32)

        @pl.when(h == NH - 1)
        def _():
            @pl.when(e < ne)
            def _():
                out_ref[0] = (acc_ref[...] + b2_ref[0]) * ss_ref[0]

            if zero_block:
                @pl.when(e == ne)
                def _():
                    out_ref[0] = jnp.zeros_like(out_ref[0])

    return body


def _mlp_part(xs_half, W1, b1, W2, b2, sscale, e0, ne, zero_block, init):
    ng = ne + (1 if zero_block else 0)
    eix = lambda e: e0 + jnp.minimum(e, ne - 1)
    hix = lambda e, h: jnp.where(e < ne, h, NH - 1)
    in_specs = [
        pl.BlockSpec((1, CAP, D), lambda e, h: (jnp.minimum(e, ne - 1), 0, 0)),
        pl.BlockSpec((1, D, HB), lambda e, h: (eix(e), 0, hix(e, h))),
        pl.BlockSpec((1, 1, HB), lambda e, h: (eix(e), 0, hix(e, h))),
        pl.BlockSpec((1, HB, D), lambda e, h: (eix(e), hix(e, h), 0)),
        pl.BlockSpec((1, 1, D), lambda e, h: (eix(e), 0, 0)),
        pl.BlockSpec((1, CAP, 1), lambda e, h: (eix(e), 0, 0)),
    ]
    args = [xs_half.reshape(ne, CAP, D), W1, b1.reshape(E, 1, H), W2,
            b2.reshape(E, 1, D), sscale.reshape(E, CAP, 1)]
    io_aliases = {}
    if init is not None:
        in_specs.append(pl.BlockSpec(memory_space=pl.ANY))
        args.append(init)
        io_aliases = {6: 0}
    if zero_block:
        out_map = lambda e, h: (jnp.where(e < ne, e0 + e, E), 0, 0)
    else:
        out_map = lambda e, h: (e0 + e, 0, 0)
    return pl.pallas_call(
        _make_mlp_body(ne, zero_block, init is not None),
        grid=(ng, NH),
        in_specs=in_specs,
        out_specs=pl.BlockSpec((1, CAP, D), out_map),
        out_shape=jax.ShapeDtypeStruct((E + 1, CAP, D), jnp.float32),
        scratch_shapes=[pltpu.VMEM((CAP, D), jnp.float32)],
        input_output_aliases=io_aliases,
        compiler_params=pltpu.CompilerParams(
            dimension_semantics=("arbitrary", "arbitrary"),
            vmem_limit_bytes=100 * 1024 * 1024,
        ),
    )(*args)


def kernel(x, Wg, bg, W1, b1, W2, b2):
    orig_shape = x.shape
    xf = x.reshape(T, D)

    slot, scale = _route(xf, Wg, bg)
    slot = slot.reshape(T)
    scale = scale.reshape(T)

    src, sscale = _invert(slot, scale)
    xs = _gather_rows(xf, src)
    yb = _mlp_part(xs, W1, b1, W2, b2, sscale, 0, E, True, None)
    y = _gather_rows(yb.reshape((E + 1) * CAP, D), slot)
    return y.reshape(orig_shape)


# speedup vs baseline: 1.0330x; 1.0095x over previous
import functools

import jax
import jax.numpy as jnp
from jax import lax
from jax.experimental import pallas as pl
from jax.experimental.pallas import tpu as pltpu
from jax.experimental.pallas import tpu_sc as plsc

D = 1024
H = 4096
E = 8
T = 8192
CAP = T // E
NSLOT = E * CAP
DUMP = NSLOT

BT = 2048
HB = 2048
NH = H // HB

NC = 2
NS = 16
NW = NC * NS
LANES = 16

ROWS_PER_W = T // NW
CHUNK = 32


def _route_body(x_ref, wg_ref, bg_ref, slot_ref, scale_ref, cnt_ref):
    pi = pl.program_id(0)

    @pl.when(pi == 0)
    def _():
        cnt_ref[...] = jnp.zeros((1, E), jnp.int32)

    x = x_ref[...]
    logits = lax.dot_general(
        x, wg_ref[...], (((1,), (0,)), ((), ())),
        preferred_element_type=jnp.float32,
    ) + bg_ref[...]

    m = jnp.max(logits, axis=1, keepdims=True)
    p = jnp.exp(logits - m)
    denom = jnp.sum(p, axis=1, keepdims=True)
    gate = 1.0 / denom

    idx = jnp.argmax(logits, axis=1)[:, None].astype(jnp.int32)
    lane = lax.broadcasted_iota(jnp.int32, (BT, E), 1)
    oh = (lane == idx).astype(jnp.float32)
    SB = 256
    row = lax.broadcasted_iota(jnp.int32, (SB, SB), 0)
    col = lax.broadcasted_iota(jnp.int32, (SB, SB), 1)
    ltri = (row > col).astype(jnp.bfloat16)
    offs = cnt_ref[...].astype(jnp.float32)
    ranks = []
    for k in range(BT // SB):
        ohk = oh[k * SB:(k + 1) * SB]
        csub = lax.dot_general(ltri, ohk.astype(jnp.bfloat16),
                               (((1,), (0,)), ((), ())),
                               preferred_element_type=jnp.float32)
        rk = (jnp.sum(csub * ohk, axis=1, keepdims=True)
              + jnp.sum(ohk * offs, axis=1, keepdims=True))
        ranks.append(rk)
        offs = offs + jnp.sum(ohk, axis=0, keepdims=True)
    rank = jnp.concatenate(ranks, axis=0).astype(jnp.int32)
    cnt_ref[...] = offs.astype(jnp.int32)

    keep = rank < CAP
    slot_ref[...] = jnp.where(keep, idx * CAP + rank, DUMP)
    scale_ref[...] = jnp.where(keep, gate, 0.0)


def _route(xf, Wg, bg):
    return pl.pallas_call(
        _route_body,
        grid=(T // BT,),
        in_specs=[
            pl.BlockSpec((BT, D), lambda i: (i, 0)),
            pl.BlockSpec((D, E), lambda i: (0, 0)),
            pl.BlockSpec((1, E), lambda i: (0, 0)),
        ],
        out_specs=[
            pl.BlockSpec((BT, 1), lambda i: (i, 0)),
            pl.BlockSpec((BT, 1), lambda i: (i, 0)),
        ],
        out_shape=[
            jax.ShapeDtypeStruct((T, 1), jnp.int32),
            jax.ShapeDtypeStruct((T, 1), jnp.float32),
        ],
        scratch_shapes=[pltpu.VMEM((1, E), jnp.int32)],
        compiler_params=pltpu.CompilerParams(
            dimension_semantics=("arbitrary",),
        ),
    )(xf, Wg, bg.reshape(1, E))


def _invert(slot, scale):
    mesh = plsc.VectorSubcoreMesh(core_axis_name="c", subcore_axis_name="s")

    @functools.partial(
        pl.kernel,
        mesh=mesh,
        out_type=[
            jax.ShapeDtypeStruct((NSLOT,), jnp.int32),
            jax.ShapeDtypeStruct((NSLOT,), jnp.float32),
        ],
        scratch_types=[
            pltpu.VMEM((T,), jnp.int32),
            pltpu.VMEM((T,), jnp.float32),
            pltpu.VMEM((NSLOT,), jnp.int32),
            pltpu.VMEM((NSLOT,), jnp.float32),
        ],
        compiler_params=pltpu.CompilerParams(needs_layout_passes=False),
    )
    def k(slot_hbm, scale_hbm, src_hbm, sscale_hbm, slot_v, scale_v,
          src_v, sscale_v):
        wid = lax.axis_index("c") * NS + lax.axis_index("s")

        @pl.when(wid == 0)
        def _():
            pltpu.sync_copy(slot_hbm, slot_v)
            pltpu.sync_copy(scale_hbm, scale_v)
            zero_i = jnp.zeros((LANES,), jnp.int32)
            zero_f = jnp.zeros((LANES,), jnp.float32)

            UNROLL = 4

            def init(i, _):
                for u in range(UNROLL):
                    src_v[pl.ds((i * UNROLL + u) * LANES, LANES)] = zero_i
                    sscale_v[pl.ds((i * UNROLL + u) * LANES, LANES)] = zero_f
                return 0

            lax.fori_loop(0, NSLOT // LANES // UNROLL, init, 0)

            tbase = lax.iota(jnp.int32, LANES)

            def scat(i, _):
                for u in range(UNROLL):
                    j = i * UNROLL + u
                    s = slot_v[pl.ds(j * LANES, LANES)]
                    tok = tbase + j * LANES
                    plsc.store_scatter(src_v, [s], tok, mask=s < NSLOT)
                return 0

            lax.fori_loop(0, T // LANES // UNROLL, scat, 0)

            def gath(i, _):
                for u in range(UNROLL):
                    j = i * UNROLL + u
                    sv = plsc.load_gather(scale_v,
                                          [src_v[pl.ds(j * LANES, LANES)]])
                    sscale_v[pl.ds(j * LANES, LANES)] = sv
                return 0

            lax.fori_loop(0, NSLOT // LANES // UNROLL, gath, 0)

            pltpu.sync_copy(src_v, src_hbm)
            pltpu.sync_copy(sscale_v, sscale_hbm)

    return k(slot, scale)


def _gather_rows(table, idx, base=0, nrows=None):
    n = nrows if nrows is not None else idx.shape[0]
    mesh = plsc.VectorSubcoreMesh(core_axis_name="c", subcore_axis_name="s")

    rpw = n // NW
    nchunk = rpw // CHUNK

    @functools.partial(
        pl.kernel,
        mesh=mesh,
        out_type=jax.ShapeDtypeStruct((n, D), jnp.float32),
        scratch_types=[
            pltpu.VMEM((2, CHUNK), jnp.int32),
            pltpu.VMEM((2, CHUNK, D), jnp.float32),
            [pltpu.SemaphoreType.DMA] * 2,
            [pltpu.SemaphoreType.DMA] * 2,
        ],
        compiler_params=pltpu.CompilerParams(needs_layout_passes=False),
    )
    def k(table_hbm, idx_hbm, out_hbm, idx_v, rows_v, gsem, wsem):
        wid = lax.axis_index("c") * NS + lax.axis_index("s")

        def start_gather(c):
            b = c % 2
            lb = wid * rpw + c * CHUNK
            pltpu.sync_copy(idx_hbm.at[pl.ds(base + lb, CHUNK)], idx_v.at[b])
            return pltpu.async_copy(table_hbm.at[idx_v.at[b]], rows_v.at[b],
                                    gsem[b])

        gh = [None, None]
        wh = [None, None]
        gh[0] = start_gather(0)
        for c in range(nchunk):
            b = c % 2
            gh[b].wait()
            if c + 1 < nchunk:
                if wh[(c + 1) % 2] is not None:
                    wh[(c + 1) % 2].wait()
                gh[(c + 1) % 2] = start_gather(c + 1)
            lb = wid * rpw + c * CHUNK
            wh[b] = pltpu.async_copy(rows_v.at[b],
                                     out_hbm.at[pl.ds(lb, CHUNK)], wsem[b])
        wh[(nchunk - 1) % 2].wait()
        if nchunk >= 2:
            wh[(nchunk - 2) % 2].wait()

    return k(table, idx)


def _make_mlp_body(ne, zero_block, aliased):
    def body(xs_ref, w1_ref, b1_ref, w2_ref, b2_ref, ss_ref, *rest):
        if aliased:
            _, out_ref, acc_ref = rest
        else:
            out_ref, acc_ref = rest
        e = pl.program_id(0)
        h = pl.program_id(1)

        @pl.when(jnp.logical_and(e < ne, h == 0))
        def _():
            acc_ref[...] = jnp.zeros_like(acc_ref)

        @pl.when(e < ne)
        def _():
            xb = xs_ref[0].astype(jnp.bfloat16)
            hpre = lax.dot_general(
                xb, w1_ref[0].astype(jnp.bfloat16), (((1,), (0,)), ((), ())),
                preferred_element_type=jnp.float32) + b1_ref[0]
            hrelu = jnp.maximum(hpre, 0.0).astype(jnp.bfloat16)
            acc_ref[...] += lax.dot_general(
                hrelu, w2_ref[0].astype(jnp.bfloat16), (((1,), (0,)), ((), ())),
                preferred_element_type=jnp.float32)

        @pl.when(h == NH - 1)
        def _():
            @pl.when(e < ne)
            def _():
                out_ref[0] = (acc_ref[...] + b2_ref[0]) * ss_ref[0]

            if zero_block:
                @pl.when(e == ne)
                def _():
                    out_ref[0] = jnp.zeros_like(out_ref[0])

    return body


def _mlp_part(xs_half, W1, b1, W2, b2, sscale, e0, ne, zero_block, init):
    ng = ne + (1 if zero_block else 0)
    eix = lambda e: e0 + jnp.minimum(e, ne - 1)
    hix = lambda e, h: jnp.where(e < ne, h, NH - 1)
    in_specs = [
        pl.BlockSpec((1, CAP, D), lambda e, h: (jnp.minimum(e, ne - 1), 0, 0)),
        pl.BlockSpec((1, D, HB), lambda e, h: (eix(e), 0, hix(e, h))),
        pl.BlockSpec((1, 1, HB), lambda e, h: (eix(e), 0, hix(e, h))),
        pl.BlockSpec((1, HB, D), lambda e, h: (eix(e), hix(e, h), 0)),
        pl.BlockSpec((1, 1, D), lambda e, h: (eix(e), 0, 0)),
        pl.BlockSpec((1, CAP, 1), lambda e, h: (eix(e), 0, 0)),
    ]
    args = [xs_half.reshape(ne, CAP, D), W1, b1.reshape(E, 1, H), W2,
            b2.reshape(E, 1, D), sscale.reshape(E, CAP, 1)]
    io_aliases = {}
    if init is not None:
        in_specs.append(pl.BlockSpec(memory_space=pl.ANY))
        args.append(init)
        io_aliases = {6: 0}
    if zero_block:
        out_map = lambda e, h: (jnp.where(e < ne, e0 + e, E), 0, 0)
    else:
        out_map = lambda e, h: (e0 + e, 0, 0)
    return pl.pallas_call(
        _make_mlp_body(ne, zero_block, init is not None),
        grid=(ng, NH),
        in_specs=in_specs,
        out_specs=pl.BlockSpec((1, CAP, D), out_map),
        out_shape=jax.ShapeDtypeStruct((E + 1, CAP, D), jnp.float32),
        scratch_shapes=[pltpu.VMEM((CAP, D), jnp.float32)],
        input_output_aliases=io_aliases,
        compiler_params=pltpu.CompilerParams(
            dimension_semantics=("arbitrary", "arbitrary"),
            vmem_limit_bytes=100 * 1024 * 1024,
        ),
    )(*args)


def kernel(x, Wg, bg, W1, b1, W2, b2):
    orig_shape = x.shape
    xf = x.reshape(T, D)

    slot, scale = _route(xf, Wg, bg)
    slot = slot.reshape(T)
    scale = scale.reshape(T)

    src, sscale = _invert(slot, scale)
    xs = _gather_rows(xf, src)
    yb = _mlp_part(xs, W1, b1, W2, b2, sscale, 0, E, True, None)
    y = _gather_rows(yb.reshape((E + 1) * CAP, D), slot)
    return y.reshape(orig_shape)
